# baseline (speedup 1.0000x reference)
import jax
import jax.numpy as jnp
from jax.experimental import pallas as pl


def _tconv(x, W, b):
    y = jax.lax.conv_general_dilated(x, W[:, :, :, None], (1, 1), 'VALID', dimension_numbers=('NCHW', 'OIHW', 'NCHW'))
    return y + b[None, :, None, None]


def _gconv(x, src, dst, ew, W, b):
    m = x[..., src] * ew[None, None, None, :]
    agg = jnp.zeros_like(x).at[..., dst].add(m)
    return jnp.einsum('bctn,co->botn', agg, W) + b[None, :, None, None]


def _ln(x, g, bb):
    xp = jnp.transpose(x, (0, 2, 3, 1))
    mu = xp.mean(axis=(2, 3), keepdims=True)
    var = xp.var(axis=(2, 3), keepdims=True)
    xn = (xp - mu) / jnp.sqrt(var + 1e-5) * g[None, None] + bb[None, None]
    return jnp.transpose(xn, (0, 3, 1, 2))


def kernel(x_steam, x_e, t, edge_index, steam_weight, steam_d, e_weight, e_d,
           steam_emb_W, steam_emb_b, e_emb_W, e_emb_b,
           edge_W1, edge_b1, edge_W2, edge_b2, virtual_edge,
           s1_tc1_W, s1_tc1_b, s1_gc_W, s1_gc_b, s1_tc2_W, s1_tc2_b, s1_ln_g, s1_ln_b,
           s2_tc1_W, s2_tc1_b, s2_gc_W, s2_gc_b, s2_tc2_W, s2_tc2_b, s2_ln_g, s2_ln_b,
           out_tc_W, out_tc_b, out_ln_g, out_ln_b,
           out_fc1_W, out_fc1_b, out_fc2_W, out_fc2_b):
    src = edge_index[0]
    dst = edge_index[1]

    def emb(w, d, scale):
        f = jnp.concatenate([w, d / scale], axis=-1)
        h = jax.nn.relu(f @ edge_W1 + edge_b1)
        return jax.nn.sigmoid(h @ edge_W2 + edge_b2)

    sf = emb(steam_weight, steam_d, 100.0)
    ef = emb(e_weight, e_d, 1000.0)
    ew = jnp.concatenate([sf, ef, virtual_edge], axis=0)[:, 0]
    xs = jnp.einsum('bitn,ci->bctn', x_steam, steam_emb_W) + steam_emb_b[None, :, None, None]
    xe = jnp.einsum('bitn,ci->bctn', x_e, e_emb_W) + e_emb_b[None, :, None, None]
    x = jnp.concatenate([xs, xe], axis=-1)

    def st_block(x, tc1_W, tc1_b, gc_W, gc_b, tc2_W, tc2_b, ln_g, ln_b):
        x = jax.nn.relu(_tconv(x, tc1_W, tc1_b))
        x = jax.nn.relu(_gconv(x, src, dst, ew, gc_W, gc_b))
        x = jax.nn.relu(_tconv(x, tc2_W, tc2_b))
        return _ln(x, ln_g, ln_b)

    x = st_block(x, s1_tc1_W, s1_tc1_b, s1_gc_W, s1_gc_b, s1_tc2_W, s1_tc2_b, s1_ln_g, s1_ln_b)
    x = st_block(x, s2_tc1_W, s2_tc1_b, s2_gc_W, s2_gc_b, s2_tc2_W, s2_tc2_b, s2_ln_g, s2_ln_b)
    x = _tconv(x, out_tc_W, out_tc_b)
    x = _ln(x, out_ln_g, out_ln_b)
    xp = jnp.transpose(x, (0, 2, 3, 1))
    xp = jax.nn.relu(xp @ out_fc1_W + out_fc1_b)
    xp = xp @ out_fc2_W + out_fc2_b
    return jnp.transpose(xp, (0, 3, 1, 2))


# speedup vs baseline: 19.9207x; 19.9207x over previous
import functools

import jax
import jax.numpy as jnp
from jax import lax
from jax.experimental import pallas as pl
from jax.experimental.pallas import tpu as pltpu
from jax.experimental.pallas import tpu_sc as plsc

N = 10000
E = 160001
NC, NS = 2, 16
NW = NC * NS
CH = 128
CPT = 40
EPAD = NW * CPT * CH
NPS = 632
NP = NS * NPS

_F32 = jnp.float32
_BF16 = jnp.bfloat16
_VMEM_LIM = pltpu.CompilerParams(vmem_limit_bytes=62 * 1024 * 1024)


def _b(x):
    return x.astype(_BF16).astype(_F32)


def _bdot(a, b):
    return jnp.dot(a.astype(_BF16), b.astype(_BF16),
                   preferred_element_type=_F32)


def _ew_body(sw, sd, ewt, edt, eW1, eb1, eW2, eb2, ve, ew_out):
    def emb(w, d, scale):
        wb = _b(w)
        db = _b(d / scale)
        acc = jnp.full((625, 128), eb2[0], _F32)
        for j in range(8):
            w1a = eW1[0, j].astype(_BF16).astype(_F32)
            w1b = eW1[1, j].astype(_BF16).astype(_F32)
            h = jnp.maximum(wb * w1a + db * w1b + eb1[j], 0.0)
            acc = acc + _b(h) * eW2[j].astype(_BF16).astype(_F32)
        return 1.0 / (1.0 + jnp.exp(-acc))

    sf = emb(sw[...], sd[...], 100.0)
    ef = emb(ewt[...], edt[...], 1000.0)
    lane = lax.broadcasted_iota(jnp.int32, (1, 128), 1)
    verow = jnp.where(lane == 0, ve[0], 0.0).astype(_F32)
    ew_out[...] = jnp.concatenate(
        [sf, ef, verow, jnp.zeros((29, 128), _F32)], axis=0)


def _run_ew(sw, sd, ewt, edt, eW1, eb1, eW2, eb2, ve):
    vmem = pl.BlockSpec(memory_space=pltpu.VMEM)
    smem = pl.BlockSpec(memory_space=pltpu.SMEM)
    return pl.pallas_call(
        _ew_body,
        out_shape=jax.ShapeDtypeStruct((EPAD // 128, 128), _F32),
        in_specs=[vmem] * 4 + [smem] * 5,
        out_specs=vmem,
    )(sw, sd, ewt, edt, eW1, eb1, eW2, eb2, ve)


def _tc1_body(xsT, xeT, semW, semb, eemW, eemb, W1m, b1r,
              ra_out, rb_out, rc_out, Ebuf):
    semWb = semW[...]
    eemWb = eemW[...]
    semb_ = semb[...]
    eemb_ = eemb[...]
    for t in range(12):
        xs_c = xsT[:, t:t + 1]
        xe_c = xeT[:, t:t + 1]
        Ebuf[0:5000, 32 * t:32 * t + 32] = xs_c * semWb + semb_
        Ebuf[5000:N, 32 * t:32 * t + 32] = xe_c * eemWb + eemb_
    W1m_ = W1m[...]
    b1_ = b1r[...]
    rc_out[:, 64:128] = jnp.zeros((N, 64), _F32)
    for t0 in range(10):
        X3 = Ebuf[:, 32 * t0:32 * t0 + 96]
        r = jnp.maximum(_bdot(X3, W1m_) + b1_, 0.0)
        if t0 < 4:
            ra_out[:, 32 * t0:32 * t0 + 32] = r
        elif t0 < 8:
            rb_out[:, 32 * (t0 - 4):32 * (t0 - 4) + 32] = r
        else:
            rc_out[:, 32 * (t0 - 8):32 * (t0 - 8) + 32] = r


def _run_tc1(xsT, xeT, semW, semb, eemW, eemb, W1m, b1r):
    vmem = pl.BlockSpec(memory_space=pltpu.VMEM)
    return pl.pallas_call(
        _tc1_body,
        compiler_params=_VMEM_LIM,
        out_shape=(jax.ShapeDtypeStruct((N, 128), _F32),
                   jax.ShapeDtypeStruct((N, 128), _F32),
                   jax.ShapeDtypeStruct((N, 128), _F32)),
        in_specs=[vmem] * 8,
        out_specs=(vmem, vmem, vmem),
        scratch_shapes=[pltpu.VMEM((N, 384), _F32)],
    )(xsT, xeT, semW, semb, eemW, eemb, W1m, b1r)


@functools.lru_cache(maxsize=None)
def _gconv_sc(F):
    FV = F // 16
    mesh = plsc.VectorSubcoreMesh(core_axis_name="c", subcore_axis_name="s")

    @functools.partial(
        pl.kernel,
        out_type=jax.ShapeDtypeStruct((NC * NP, F), _F32),
        mesh=mesh,
        scratch_types=[
            pltpu.VMEM((CH,), jnp.int32),
            pltpu.VMEM((CH,), jnp.int32),
            pltpu.VMEM((CPT * CH,), _F32),
            pltpu.VMEM((CH, F), _F32),
            pltpu.VMEM_SHARED((NP, F), _F32),
            pltpu.SemaphoreType.DMA,
        ],
    )
    def k(xp_hbm, src_hbm, dst_hbm, ew_hbm, zeros_hbm, out_hbm,
          sidx, didx, ewv, rows, acc, sem):
        ci = lax.axis_index("c")
        si = lax.axis_index("s")
        wid = ci * NS + si

        pltpu.sync_copy(ew_hbm.at[pl.ds(wid * (CPT * CH), CPT * CH)], ewv)

        pltpu.sync_copy(zeros_hbm, acc.at[pl.ds(si * NPS, NPS)])
        plsc.subcore_barrier()

        def chunk(g, carry):
            base = (wid * CPT + g) * CH
            pltpu.sync_copy(src_hbm.at[pl.ds(base, CH)], sidx)
            pltpu.sync_copy(dst_hbm.at[pl.ds(base, CH)], didx)
            pltpu.async_copy(xp_hbm.at[sidx], rows, sem).wait()

            def scale(j, c2):
                wvec = ewv[pl.ds(g * CH + j * 16, 16)]
                for e16 in range(16):
                    w = jnp.broadcast_to(wvec[e16], (16,))
                    e = j * 16 + e16
                    for kk in range(FV):
                        rows[e, pl.ds(16 * kk, 16)] = (
                            rows[e, pl.ds(16 * kk, 16)] * w)
                return c2

            lax.fori_loop(0, CH // 16, scale, 0)
            pltpu.sync_copy(rows, acc.at[didx], add=True)
            return carry

        lax.fori_loop(0, CPT, chunk, 0)
        plsc.subcore_barrier()

        pltpu.sync_copy(acc.at[pl.ds(si * NPS, NPS)],
                        out_hbm.at[pl.ds(ci * NP + si * NPS, NPS)])

    return k


def _tc2a_body(pa0, pa1, pb0, pb1, pc0, pc1, gcW, gcb, g_out):
    gcW_ = gcW[...]
    gcb_ = gcb[...]
    pairs = ((pa0, pa1), (pb0, pb1), (pc0, pc1))
    for t0 in range(10):
        p0, p1 = pairs[t0 // 4]
        c = 32 * (t0 % 4)
        agg = p0[:, c:c + 32] + p1[:, c:c + 32]
        g_out[:, 16 * t0:16 * t0 + 16] = jnp.maximum(
            _bdot(agg, gcW_) + gcb_, 0.0)


def _run_tc2a(pa0, pa1, pb0, pb1, pc0, pc1, gcW, gcb):
    vmem = pl.BlockSpec(memory_space=pltpu.VMEM)
    return pl.pallas_call(
        _tc2a_body,
        compiler_params=_VMEM_LIM,
        out_shape=jax.ShapeDtypeStruct((N, 160), _F32),
        in_specs=[vmem] * 8,
        out_specs=vmem,
    )(pa0, pa1, pb0, pb1, pc0, pc1, gcW, gcb)


def _tc2b_body(G, Wf, bf, ln_g, ln_b, W2f, b2f, md_out, me_out, Lbuf):
    Wf_ = Wf[...]
    bf_ = bf[...]
    g_ = ln_g[...]
    b_ = ln_b[...]
    for t0 in range(8):
        H = jnp.maximum(_bdot(G[:, 16 * t0:16 * t0 + 48], Wf_) + bf_, 0.0)
        mu = jnp.mean(H)
        var = jnp.mean(H * H) - mu * mu
        Lbuf[:, 32 * t0:32 * t0 + 32] = (H - mu) * lax.rsqrt(var + 1e-5) * g_ + b_
    W2f_ = W2f[...]
    b2f_ = b2f[...]
    me_out[:, 64:128] = jnp.zeros((N, 64), _F32)
    for t0 in range(6):
        M = jnp.maximum(_bdot(Lbuf[:, 32 * t0:32 * t0 + 96], W2f_) + b2f_, 0.0)
        if t0 < 4:
            md_out[:, 32 * t0:32 * t0 + 32] = M
        else:
            me_out[:, 32 * (t0 - 4):32 * (t0 - 4) + 32] = M


def _run_tc2b(G, Wf, bf, ln_g, ln_b, W2f, b2f):
    vmem = pl.BlockSpec(memory_space=pltpu.VMEM)
    return pl.pallas_call(
        _tc2b_body,
        compiler_params=_VMEM_LIM,
        out_shape=(jax.ShapeDtypeStruct((N, 128), _F32),
                   jax.ShapeDtypeStruct((N, 128), _F32)),
        in_specs=[vmem] * 7,
        out_specs=(vmem, vmem),
        scratch_shapes=[pltpu.VMEM((N, 256), _F32)],
    )(G, Wf, bf, ln_g, ln_b, W2f, b2f)


def _tc3a_body(pd0, pd1, pe0, pe1, gcW, gcb, g_out):
    gcW_ = gcW[...]
    gcb_ = gcb[...]
    pairs = ((pd0, pd1), (pe0, pe1))
    for t0 in range(6):
        p0, p1 = pairs[t0 // 4]
        c = 32 * (t0 % 4)
        agg = p0[:, c:c + 32] + p1[:, c:c + 32]
        g_out[:, 16 * t0:16 * t0 + 16] = jnp.maximum(
            _bdot(agg, gcW_) + gcb_, 0.0)


def _run_tc3a(pd0, pd1, pe0, pe1, gcW, gcb):
    vmem = pl.BlockSpec(memory_space=pltpu.VMEM)
    return pl.pallas_call(
        _tc3a_body,
        compiler_params=_VMEM_LIM,
        out_shape=jax.ShapeDtypeStruct((N, 96), _F32),
        in_specs=[vmem] * 6,
        out_specs=vmem,
    )(pd0, pd1, pe0, pe1, gcW, gcb)


def _tc3b_body(G2, Wf, bf, ln_g, ln_b, l_out):
    Wf_ = Wf[...]
    bf_ = bf[...]
    g_ = ln_g[...]
    b_ = ln_b[...]
    for t0 in range(4):
        H = jnp.maximum(_bdot(G2[:, 16 * t0:16 * t0 + 48], Wf_) + bf_, 0.0)
        mu = jnp.mean(H)
        var = jnp.mean(H * H) - mu * mu
        l_out[:, 32 * t0:32 * t0 + 32] = (H - mu) * lax.rsqrt(var + 1e-5) * g_ + b_


def _run_tc3b(G2, Wf, bf, ln_g, ln_b):
    vmem = pl.BlockSpec(memory_space=pltpu.VMEM)
    return pl.pallas_call(
        _tc3b_body,
        compiler_params=_VMEM_LIM,
        out_shape=jax.ShapeDtypeStruct((N, 128), _F32),
        in_specs=[vmem] * 5,
        out_specs=vmem,
    )(G2, Wf, bf, ln_g, ln_b)


def _tc3c_body(L, Wof, bof, og, ob, W1, b1, W2, b2, y_out):
    Y = _bdot(L[...], Wof[...]) + bof[...]
    mu = jnp.mean(Y)
    var = jnp.mean(Y * Y) - mu * mu
    Z = (Y - mu) * lax.rsqrt(var + 1e-5) * og[...] + ob[...]
    F1 = jnp.maximum(_bdot(Z, W1[...]) + b1[...], 0.0)
    y_out[...] = jnp.sum(_b(F1) * _b(W2[...]), axis=1, keepdims=True) + b2[...]


def _run_tc3c(L, Wof, bof, og, ob, W1, b1, W2, b2):
    vmem = pl.BlockSpec(memory_space=pltpu.VMEM)
    return pl.pallas_call(
        _tc3c_body,
        compiler_params=_VMEM_LIM,
        out_shape=jax.ShapeDtypeStruct((N, 1), _F32),
        in_specs=[vmem] * 9,
        out_specs=vmem,
    )(L, Wof, bof, og, ob, W1, b1, W2, b2)


def kernel(x_steam, x_e, t, edge_index, steam_weight, steam_d, e_weight, e_d,
           steam_emb_W, steam_emb_b, e_emb_W, e_emb_b,
           edge_W1, edge_b1, edge_W2, edge_b2, virtual_edge,
           s1_tc1_W, s1_tc1_b, s1_gc_W, s1_gc_b, s1_tc2_W, s1_tc2_b, s1_ln_g, s1_ln_b,
           s2_tc1_W, s2_tc1_b, s2_gc_W, s2_gc_b, s2_tc2_W, s2_tc2_b, s2_ln_g, s2_ln_b,
           out_tc_W, out_tc_b, out_ln_g, out_ln_b,
           out_fc1_W, out_fc1_b, out_fc2_W, out_fc2_b):
    xsT = x_steam.reshape(12, 5000).T
    xeT = x_e.reshape(12, 5000).T
    sw = steam_weight.reshape(625, 128)
    sd = steam_d.reshape(625, 128)
    ewt = e_weight.reshape(625, 128)
    edt = e_d.reshape(625, 128)
    W1m = jnp.transpose(s1_tc1_W, (2, 1, 0)).reshape(96, 32)
    ei_pad = jnp.concatenate(
        [edge_index, jnp.zeros((2, EPAD - E), jnp.int32)], axis=1)
    src1d = ei_pad[0]
    dst1d = ei_pad[1]
    zeros128 = jnp.zeros((NPS, 128), _F32)

    ew1d = _run_ew(
        sw, sd, ewt, edt,
        edge_W1, edge_b1, edge_W2.reshape(8), edge_b2,
        virtual_edge.reshape(1)).reshape(EPAD)

    RA, RB, RC = _run_tc1(
        xsT, xeT,
        steam_emb_W.T, steam_emb_b.reshape(1, 32),
        e_emb_W.T, e_emb_b.reshape(1, 32),
        W1m, s1_tc1_b.reshape(1, 32))

    sc = _gconv_sc(128)
    pA = sc(RA, src1d, dst1d, ew1d, zeros128)
    pB = sc(RB, src1d, dst1d, ew1d, zeros128)
    pC = sc(RC, src1d, dst1d, ew1d, zeros128)

    G = _run_tc2a(pA[:N], pA[NP:NP + N], pB[:N], pB[NP:NP + N],
                  pC[:N], pC[NP:NP + N], s1_gc_W, s1_gc_b.reshape(1, 16))

    MD, ME = _run_tc2b(
        G,
        jnp.transpose(s1_tc2_W, (2, 1, 0)).reshape(48, 32),
        s1_tc2_b.reshape(1, 32), s1_ln_g, s1_ln_b,
        jnp.transpose(s2_tc1_W, (2, 1, 0)).reshape(96, 32),
        s2_tc1_b.reshape(1, 32))

    pD = sc(MD, src1d, dst1d, ew1d, zeros128)
    pE = sc(ME, src1d, dst1d, ew1d, zeros128)

    G2 = _run_tc3a(pD[:N], pD[NP:NP + N], pE[:N], pE[NP:NP + N],
                   s2_gc_W, s2_gc_b.reshape(1, 16))

    L = _run_tc3b(
        G2,
        jnp.transpose(s2_tc2_W, (2, 1, 0)).reshape(48, 32),
        s2_tc2_b.reshape(1, 32), s2_ln_g, s2_ln_b)

    y = _run_tc3c(
        L,
        jnp.transpose(out_tc_W, (2, 1, 0)).reshape(128, 64),
        out_tc_b.reshape(1, 64), out_ln_g, out_ln_b,
        out_fc1_W, out_fc1_b.reshape(1, 64),
        out_fc2_W.reshape(1, 64), out_fc2_b.reshape(1, 1))

    return y.reshape(1, 1, 1, N)


# speedup vs baseline: 25.1538x; 1.2627x over previous
import functools

import jax
import jax.numpy as jnp
from jax import lax
from jax.experimental import pallas as pl
from jax.experimental.pallas import tpu as pltpu
from jax.experimental.pallas import tpu_sc as plsc

N = 10000
E = 160001
NC, NS = 2, 16
NW = NC * NS
CH = 128
CPT = 40
EPAD = NW * CPT * CH
NPS = 632
NP = NS * NPS

_F32 = jnp.float32
_BF16 = jnp.bfloat16
_VMEM_LIM = pltpu.CompilerParams(vmem_limit_bytes=62 * 1024 * 1024)


def _b(x):
    return x.astype(_BF16).astype(_F32)


def _bdot(a, b):
    return jnp.dot(a.astype(_BF16), b.astype(_BF16),
                   preferred_element_type=_F32)


def _ew_body(sw, sd, ewt, edt, eW1, eb1, eW2, eb2, ve, ew_out):
    def emb(w, d, scale):
        wb = _b(w)
        db = _b(d / scale)
        acc = jnp.full((625, 128), eb2[0], _F32)
        for j in range(8):
            w1a = eW1[0, j].astype(_BF16).astype(_F32)
            w1b = eW1[1, j].astype(_BF16).astype(_F32)
            h = jnp.maximum(wb * w1a + db * w1b + eb1[j], 0.0)
            acc = acc + _b(h) * eW2[j].astype(_BF16).astype(_F32)
        return 1.0 / (1.0 + jnp.exp(-acc))

    sf = emb(sw[...], sd[...], 100.0)
    ef = emb(ewt[...], edt[...], 1000.0)
    lane = lax.broadcasted_iota(jnp.int32, (1, 128), 1)
    verow = jnp.where(lane == 0, ve[0], 0.0).astype(_F32)
    ew_out[...] = jnp.concatenate(
        [sf, ef, verow, jnp.zeros((29, 128), _F32)], axis=0)


def _run_ew(sw, sd, ewt, edt, eW1, eb1, eW2, eb2, ve):
    vmem = pl.BlockSpec(memory_space=pltpu.VMEM)
    smem = pl.BlockSpec(memory_space=pltpu.SMEM)
    return pl.pallas_call(
        _ew_body,
        out_shape=jax.ShapeDtypeStruct((EPAD // 128, 128), _F32),
        in_specs=[vmem] * 4 + [smem] * 5,
        out_specs=vmem,
    )(sw, sd, ewt, edt, eW1, eb1, eW2, eb2, ve)


def _tc1_body(xsT, xeT, semW, semb, eemW, eemb, W1m, b1r,
              ra_out, rb_out, rc_out, Ebuf):
    semWb = semW[...]
    eemWb = eemW[...]
    semb_ = semb[...]
    eemb_ = eemb[...]
    for t in range(12):
        xs_c = xsT[:, t:t + 1]
        xe_c = xeT[:, t:t + 1]
        Ebuf[0:5000, 32 * t:32 * t + 32] = xs_c * semWb + semb_
        Ebuf[5000:N, 32 * t:32 * t + 32] = xe_c * eemWb + eemb_
    W1m_ = W1m[...]
    b1_ = b1r[...]
    rc_out[:, 64:128] = jnp.zeros((N, 64), _F32)
    for t0 in range(10):
        X3 = Ebuf[:, 32 * t0:32 * t0 + 96]
        r = jnp.maximum(_bdot(X3, W1m_) + b1_, 0.0)
        if t0 < 4:
            ra_out[:, 32 * t0:32 * t0 + 32] = r
        elif t0 < 8:
            rb_out[:, 32 * (t0 - 4):32 * (t0 - 4) + 32] = r
        else:
            rc_out[:, 32 * (t0 - 8):32 * (t0 - 8) + 32] = r


def _run_tc1(xsT, xeT, semW, semb, eemW, eemb, W1m, b1r):
    vmem = pl.BlockSpec(memory_space=pltpu.VMEM)
    return pl.pallas_call(
        _tc1_body,
        compiler_params=_VMEM_LIM,
        out_shape=(jax.ShapeDtypeStruct((N, 128), _F32),
                   jax.ShapeDtypeStruct((N, 128), _F32),
                   jax.ShapeDtypeStruct((N, 128), _F32)),
        in_specs=[vmem] * 8,
        out_specs=(vmem, vmem, vmem),
        scratch_shapes=[pltpu.VMEM((N, 384), _F32)],
    )(xsT, xeT, semW, semb, eemW, eemb, W1m, b1r)


@functools.lru_cache(maxsize=None)
def _gconv_sc(F):
    FV = F // 16
    mesh = plsc.VectorSubcoreMesh(core_axis_name="c", subcore_axis_name="s")

    @functools.partial(
        pl.kernel,
        out_type=jax.ShapeDtypeStruct((NC * NP, F), _F32),
        mesh=mesh,
        scratch_types=[
            pltpu.VMEM((CH,), jnp.int32),
            pltpu.VMEM((CH,), jnp.int32),
            pltpu.VMEM((CH,), jnp.int32),
            pltpu.VMEM((CH,), jnp.int32),
            pltpu.VMEM((CPT * CH,), _F32),
            pltpu.VMEM((CH, F), _F32),
            pltpu.VMEM((CH, F), _F32),
            pltpu.VMEM_SHARED((NP, F), _F32),
            pltpu.SemaphoreType.DMA,
            pltpu.SemaphoreType.DMA,
        ],
    )
    def k(xp_hbm, src_hbm, dst_hbm, ew_hbm, zeros_hbm, out_hbm,
          sidx0, sidx1, didx0, didx1, ewv, rows0, rows1, acc, sem0, sem1):
        ci = lax.axis_index("c")
        si = lax.axis_index("s")
        wid = ci * NS + si

        pltpu.sync_copy(ew_hbm.at[pl.ds(wid * (CPT * CH), CPT * CH)], ewv)

        pltpu.sync_copy(zeros_hbm, acc.at[pl.ds(si * NPS, NPS)])
        plsc.subcore_barrier()

        def load_idx(g, sidx, didx):
            base = (wid * CPT + g) * CH
            pltpu.sync_copy(src_hbm.at[pl.ds(base, CH)], sidx)
            pltpu.sync_copy(dst_hbm.at[pl.ds(base, CH)], didx)

        def scale(g, rows):
            def body(j, c2):
                wvec = ewv[pl.ds(g * CH + j * 16, 16)]
                for e16 in range(16):
                    w = jnp.broadcast_to(wvec[e16], (16,))
                    e = j * 16 + e16
                    for kk in range(FV):
                        rows[e, pl.ds(16 * kk, 16)] = (
                            rows[e, pl.ds(16 * kk, 16)] * w)
                return c2

            lax.fori_loop(0, CH // 16, body, 0)

        load_idx(0, sidx0, didx0)
        pltpu.async_copy(xp_hbm.at[sidx0], rows0, sem0)

        def pair(i, carry):
            g0 = 2 * i
            load_idx(g0 + 1, sidx1, didx1)
            pltpu.async_copy(xp_hbm.at[sidx1], rows1, sem1)
            pltpu.make_async_copy(xp_hbm.at[sidx0], rows0, sem0).wait()
            scale(g0, rows0)
            pltpu.sync_copy(rows0, acc.at[didx0], add=True)

            @pl.when(i + 1 < CPT // 2)
            def _():
                load_idx(g0 + 2, sidx0, didx0)
                pltpu.async_copy(xp_hbm.at[sidx0], rows0, sem0)

            pltpu.make_async_copy(xp_hbm.at[sidx1], rows1, sem1).wait()
            scale(g0 + 1, rows1)
            pltpu.sync_copy(rows1, acc.at[didx1], add=True)
            return carry

        lax.fori_loop(0, CPT // 2, pair, 0)
        plsc.subcore_barrier()

        pltpu.sync_copy(acc.at[pl.ds(si * NPS, NPS)],
                        out_hbm.at[pl.ds(ci * NP + si * NPS, NPS)])

    return k


def _tc2a_body(pa0, pa1, pb0, pb1, pc0, pc1, gcW, gcb, g_out):
    gcW_ = gcW[...]
    gcb_ = gcb[...]
    pairs = ((pa0, pa1), (pb0, pb1), (pc0, pc1))
    for t0 in range(10):
        p0, p1 = pairs[t0 // 4]
        c = 32 * (t0 % 4)
        agg = p0[:, c:c + 32] + p1[:, c:c + 32]
        g_out[:, 16 * t0:16 * t0 + 16] = jnp.maximum(
            _bdot(agg, gcW_) + gcb_, 0.0)


def _run_tc2a(pa0, pa1, pb0, pb1, pc0, pc1, gcW, gcb):
    vmem = pl.BlockSpec(memory_space=pltpu.VMEM)
    return pl.pallas_call(
        _tc2a_body,
        compiler_params=_VMEM_LIM,
        out_shape=jax.ShapeDtypeStruct((N, 160), _F32),
        in_specs=[vmem] * 8,
        out_specs=vmem,
    )(pa0, pa1, pb0, pb1, pc0, pc1, gcW, gcb)


def _tc2b_body(G, Wf, bf, ln_g, ln_b, W2f, b2f, md_out, me_out, Lbuf):
    Wf_ = Wf[...]
    bf_ = bf[...]
    g_ = ln_g[...]
    b_ = ln_b[...]
    for t0 in range(8):
        H = jnp.maximum(_bdot(G[:, 16 * t0:16 * t0 + 48], Wf_) + bf_, 0.0)
        mu = jnp.mean(H)
        var = jnp.mean(H * H) - mu * mu
        Lbuf[:, 32 * t0:32 * t0 + 32] = (H - mu) * lax.rsqrt(var + 1e-5) * g_ + b_
    W2f_ = W2f[...]
    b2f_ = b2f[...]
    me_out[:, 64:128] = jnp.zeros((N, 64), _F32)
    for t0 in range(6):
        M = jnp.maximum(_bdot(Lbuf[:, 32 * t0:32 * t0 + 96], W2f_) + b2f_, 0.0)
        if t0 < 4:
            md_out[:, 32 * t0:32 * t0 + 32] = M
        else:
            me_out[:, 32 * (t0 - 4):32 * (t0 - 4) + 32] = M


def _run_tc2b(G, Wf, bf, ln_g, ln_b, W2f, b2f):
    vmem = pl.BlockSpec(memory_space=pltpu.VMEM)
    return pl.pallas_call(
        _tc2b_body,
        compiler_params=_VMEM_LIM,
        out_shape=(jax.ShapeDtypeStruct((N, 128), _F32),
                   jax.ShapeDtypeStruct((N, 128), _F32)),
        in_specs=[vmem] * 7,
        out_specs=(vmem, vmem),
        scratch_shapes=[pltpu.VMEM((N, 256), _F32)],
    )(G, Wf, bf, ln_g, ln_b, W2f, b2f)


def _tc3a_body(pd0, pd1, pe0, pe1, gcW, gcb, g_out):
    gcW_ = gcW[...]
    gcb_ = gcb[...]
    pairs = ((pd0, pd1), (pe0, pe1))
    for t0 in range(6):
        p0, p1 = pairs[t0 // 4]
        c = 32 * (t0 % 4)
        agg = p0[:, c:c + 32] + p1[:, c:c + 32]
        g_out[:, 16 * t0:16 * t0 + 16] = jnp.maximum(
            _bdot(agg, gcW_) + gcb_, 0.0)


def _run_tc3a(pd0, pd1, pe0, pe1, gcW, gcb):
    vmem = pl.BlockSpec(memory_space=pltpu.VMEM)
    return pl.pallas_call(
        _tc3a_body,
        compiler_params=_VMEM_LIM,
        out_shape=jax.ShapeDtypeStruct((N, 96), _F32),
        in_specs=[vmem] * 6,
        out_specs=vmem,
    )(pd0, pd1, pe0, pe1, gcW, gcb)


def _tc3b_body(G2, Wf, bf, ln_g, ln_b, l_out):
    Wf_ = Wf[...]
    bf_ = bf[...]
    g_ = ln_g[...]
    b_ = ln_b[...]
    for t0 in range(4):
        H = jnp.maximum(_bdot(G2[:, 16 * t0:16 * t0 + 48], Wf_) + bf_, 0.0)
        mu = jnp.mean(H)
        var = jnp.mean(H * H) - mu * mu
        l_out[:, 32 * t0:32 * t0 + 32] = (H - mu) * lax.rsqrt(var + 1e-5) * g_ + b_


def _run_tc3b(G2, Wf, bf, ln_g, ln_b):
    vmem = pl.BlockSpec(memory_space=pltpu.VMEM)
    return pl.pallas_call(
        _tc3b_body,
        compiler_params=_VMEM_LIM,
        out_shape=jax.ShapeDtypeStruct((N, 128), _F32),
        in_specs=[vmem] * 5,
        out_specs=vmem,
    )(G2, Wf, bf, ln_g, ln_b)


def _tc3c_body(L, Wof, bof, og, ob, W1, b1, W2, b2, y_out):
    Y = _bdot(L[...], Wof[...]) + bof[...]
    mu = jnp.mean(Y)
    var = jnp.mean(Y * Y) - mu * mu
    Z = (Y - mu) * lax.rsqrt(var + 1e-5) * og[...] + ob[...]
    F1 = jnp.maximum(_bdot(Z, W1[...]) + b1[...], 0.0)
    y_out[...] = jnp.sum(_b(F1) * _b(W2[...]), axis=1, keepdims=True) + b2[...]


def _run_tc3c(L, Wof, bof, og, ob, W1, b1, W2, b2):
    vmem = pl.BlockSpec(memory_space=pltpu.VMEM)
    return pl.pallas_call(
        _tc3c_body,
        compiler_params=_VMEM_LIM,
        out_shape=jax.ShapeDtypeStruct((N, 1), _F32),
        in_specs=[vmem] * 9,
        out_specs=vmem,
    )(L, Wof, bof, og, ob, W1, b1, W2, b2)


def kernel(x_steam, x_e, t, edge_index, steam_weight, steam_d, e_weight, e_d,
           steam_emb_W, steam_emb_b, e_emb_W, e_emb_b,
           edge_W1, edge_b1, edge_W2, edge_b2, virtual_edge,
           s1_tc1_W, s1_tc1_b, s1_gc_W, s1_gc_b, s1_tc2_W, s1_tc2_b, s1_ln_g, s1_ln_b,
           s2_tc1_W, s2_tc1_b, s2_gc_W, s2_gc_b, s2_tc2_W, s2_tc2_b, s2_ln_g, s2_ln_b,
           out_tc_W, out_tc_b, out_ln_g, out_ln_b,
           out_fc1_W, out_fc1_b, out_fc2_W, out_fc2_b):
    xsT = x_steam.reshape(12, 5000).T
    xeT = x_e.reshape(12, 5000).T
    sw = steam_weight.reshape(625, 128)
    sd = steam_d.reshape(625, 128)
    ewt = e_weight.reshape(625, 128)
    edt = e_d.reshape(625, 128)
    W1m = jnp.transpose(s1_tc1_W, (2, 1, 0)).reshape(96, 32)
    ei_pad = jnp.concatenate(
        [edge_index, jnp.zeros((2, EPAD - E), jnp.int32)], axis=1)
    src1d = ei_pad[0]
    dst1d = ei_pad[1]
    zeros128 = jnp.zeros((NPS, 128), _F32)

    ew1d = _run_ew(
        sw, sd, ewt, edt,
        edge_W1, edge_b1, edge_W2.reshape(8), edge_b2,
        virtual_edge.reshape(1)).reshape(EPAD)

    RA, RB, RC = _run_tc1(
        xsT, xeT,
        steam_emb_W.T, steam_emb_b.reshape(1, 32),
        e_emb_W.T, e_emb_b.reshape(1, 32),
        W1m, s1_tc1_b.reshape(1, 32))

    sc = _gconv_sc(128)
    pA = sc(RA, src1d, dst1d, ew1d, zeros128)
    pB = sc(RB, src1d, dst1d, ew1d, zeros128)
    pC = sc(RC, src1d, dst1d, ew1d, zeros128)

    G = _run_tc2a(pA[:N], pA[NP:NP + N], pB[:N], pB[NP:NP + N],
                  pC[:N], pC[NP:NP + N], s1_gc_W, s1_gc_b.reshape(1, 16))

    MD, ME = _run_tc2b(
        G,
        jnp.transpose(s1_tc2_W, (2, 1, 0)).reshape(48, 32),
        s1_tc2_b.reshape(1, 32), s1_ln_g, s1_ln_b,
        jnp.transpose(s2_tc1_W, (2, 1, 0)).reshape(96, 32),
        s2_tc1_b.reshape(1, 32))

    pD = sc(MD, src1d, dst1d, ew1d, zeros128)
    pE = sc(ME, src1d, dst1d, ew1d, zeros128)

    G2 = _run_tc3a(pD[:N], pD[NP:NP + N], pE[:N], pE[NP:NP + N],
                   s2_gc_W, s2_gc_b.reshape(1, 16))

    L = _run_tc3b(
        G2,
        jnp.transpose(s2_tc2_W, (2, 1, 0)).reshape(48, 32),
        s2_tc2_b.reshape(1, 32), s2_ln_g, s2_ln_b)

    y = _run_tc3c(
        L,
        jnp.transpose(out_tc_W, (2, 1, 0)).reshape(128, 64),
        out_tc_b.reshape(1, 64), out_ln_g, out_ln_b,
        out_fc1_W, out_fc1_b.reshape(1, 64),
        out_fc2_W.reshape(1, 64), out_fc2_b.reshape(1, 1))

    return y.reshape(1, 1, 1, N)
